# baseline (speedup 1.0000x reference)
import functools

import jax
import jax.numpy as jnp
from jax import lax
from jax.experimental import pallas as pl
from jax.experimental.pallas import tpu as pltpu
from jax.experimental.pallas import tpu_sc as plsc

_NCORES = 2
_NSUB = 16
_NW = _NCORES * _NSUB
_L = 16


def kernel(kspace_data, mask):
    nb, nc, nx, ny = kspace_data.shape
    n_sampled = mask.shape[2] // 2
    nrows = nb * nc * nx
    rows_per_w = nrows // _NW
    chunk_rows = 64
    nchunks = rows_per_w // chunk_rows
    ngroups = n_sampled // _L

    x_flat = kspace_data.reshape(nrows * ny)
    mask_i32 = mask.reshape(ny).astype(jnp.int32)

    mesh = plsc.VectorSubcoreMesh(
        core_axis_name="c", subcore_axis_name="s",
        num_cores=_NCORES, num_subcores=_NSUB)

    @functools.partial(
        pl.kernel,
        out_type=jax.ShapeDtypeStruct((nrows * n_sampled,), jnp.float32),
        mesh=mesh,
        scratch_types=[
            pltpu.VMEM((chunk_rows * ny,), jnp.float32),
            pltpu.VMEM((chunk_rows * n_sampled,), jnp.float32),
            pltpu.VMEM((ny,), jnp.int32),
            pltpu.VMEM((ny + _L,), jnp.int32),
            pltpu.VMEM((_L,), jnp.int32),
        ],
        compiler_params=pltpu.CompilerParams(needs_layout_passes=False),
    )
    def run(x_hbm, mask_hbm, out_hbm, in_v, out_v, mask_v, cols_v, tmp_v):
        wid = lax.axis_index("s") * _NCORES + lax.axis_index("c")
        pltpu.sync_copy(mask_hbm, mask_v)

        zero = jnp.zeros((_L,), jnp.int32)
        for g in range(ny // _L + 1):
            cols_v[pl.ds(g * _L, _L)] = zero

        lane = lax.iota(jnp.int32, _L)

        def comp_body(g, off):
            m = mask_v[pl.ds(g * _L, _L)] != 0
            mi = jnp.where(m, jnp.int32(1), jnp.int32(0))
            s = mi
            for sh in (1, 2, 4, 8):
                tmp_v[...] = s
                sv = plsc.load_gather(tmp_v, [jnp.maximum(lane - sh, 0)])
                s = s + jnp.where(lane >= sh, sv, jnp.int32(0))
            tmp_v[...] = s
            tot = plsc.load_gather(tmp_v, [jnp.full((_L,), _L - 1,
                                                    jnp.int32)])
            ids = lane + g * _L
            plsc.store_scatter(cols_v, [off + s - 1], ids, mask=m)
            return off + tot
        lax.fori_loop(0, ny // _L, comp_body, jnp.zeros((_L,), jnp.int32))

        cols = [cols_v[pl.ds(g * _L, _L)] for g in range(ngroups)]

        row0 = wid * rows_per_w

        def chunk_body(cix, _):
            base_row = row0 + cix * chunk_rows
            pltpu.sync_copy(x_hbm.at[pl.ds(base_row * ny, chunk_rows * ny)],
                            in_v)

            def row_body(r, _):
                rb = r * ny
                for g in range(ngroups):
                    v = plsc.load_gather(in_v, [cols[g] + rb])
                    out_v[pl.ds(r * n_sampled + g * _L, _L)] = v
                return 0
            lax.fori_loop(0, chunk_rows, row_body, 0)

            pltpu.sync_copy(
                out_v,
                out_hbm.at[pl.ds(base_row * n_sampled,
                                 chunk_rows * n_sampled)])
            return 0
        lax.fori_loop(0, nchunks, chunk_body, 0)

    out = run(x_flat, mask_i32)
    return out.reshape(nb, nc, nx, n_sampled)


# speedup vs baseline: 1.3974x; 1.3974x over previous
import functools

import jax
import jax.numpy as jnp
from jax import lax
from jax.experimental import pallas as pl
from jax.experimental.pallas import tpu as pltpu
from jax.experimental.pallas import tpu_sc as plsc

_NCORES = 2
_NSUB = 16
_NW = _NCORES * _NSUB
_L = 16


def kernel(kspace_data, mask):
    nb, nc, nx, ny = kspace_data.shape
    n_sampled = mask.shape[2] // 2
    nrows = nb * nc * nx
    rows_per_w = nrows // _NW
    chunk_rows = 64
    nchunks = rows_per_w // chunk_rows
    ngroups = n_sampled // _L
    assert nchunks % 2 == 0

    x_flat = kspace_data.reshape(nrows * ny)
    mask_i32 = mask.reshape(ny).astype(jnp.int32)

    mesh = plsc.VectorSubcoreMesh(
        core_axis_name="c", subcore_axis_name="s",
        num_cores=_NCORES, num_subcores=_NSUB)

    @functools.partial(
        pl.kernel,
        out_type=jax.ShapeDtypeStruct((nrows * n_sampled,), jnp.float32),
        mesh=mesh,
        scratch_types=[
            pltpu.VMEM((chunk_rows * ny,), jnp.float32),
            pltpu.VMEM((chunk_rows * ny,), jnp.float32),
            pltpu.VMEM((chunk_rows * n_sampled,), jnp.float32),
            pltpu.VMEM((chunk_rows * n_sampled,), jnp.float32),
            pltpu.VMEM((ny,), jnp.int32),
            pltpu.VMEM((ny + _L,), jnp.int32),
            pltpu.VMEM((_L,), jnp.int32),
            pltpu.SemaphoreType.DMA,
            pltpu.SemaphoreType.DMA,
            pltpu.SemaphoreType.DMA,
            pltpu.SemaphoreType.DMA,
        ],
        compiler_params=pltpu.CompilerParams(needs_layout_passes=False),
    )
    def run(x_hbm, mask_hbm, out_hbm, in_v0, in_v1, out_v0, out_v1,
            mask_v, cols_v, tmp_v, isem0, isem1, osem0, osem1):
        in_bufs = (in_v0, in_v1)
        out_bufs = (out_v0, out_v1)
        isems = (isem0, isem1)
        osems = (osem0, osem1)

        wid = lax.axis_index("s") * _NCORES + lax.axis_index("c")
        pltpu.sync_copy(mask_hbm, mask_v)

        zero = jnp.zeros((_L,), jnp.int32)
        for g in range(ny // _L + 1):
            cols_v[pl.ds(g * _L, _L)] = zero

        lane = lax.iota(jnp.int32, _L)

        def comp_body(g, off):
            m = mask_v[pl.ds(g * _L, _L)] != 0
            s = jnp.where(m, jnp.int32(1), jnp.int32(0))
            for sh in (1, 2, 4, 8):
                tmp_v[...] = s
                sv = plsc.load_gather(tmp_v, [jnp.maximum(lane - sh, 0)])
                s = s + jnp.where(lane >= sh, sv, jnp.int32(0))
            tmp_v[...] = s
            tot = plsc.load_gather(tmp_v, [jnp.full((_L,), _L - 1,
                                                    jnp.int32)])
            ids = lane + g * _L
            plsc.store_scatter(cols_v, [off + s - 1], ids, mask=m)
            return off + tot
        lax.fori_loop(0, ny // _L, comp_body, jnp.zeros((_L,), jnp.int32))

        cols = [cols_v[pl.ds(g * _L, _L)] for g in range(ngroups)]

        row0 = wid * rows_per_w
        in_words = chunk_rows * ny
        out_words = chunk_rows * n_sampled

        def in_slice(c):
            return x_hbm.at[pl.ds((row0 + c * chunk_rows) * ny, in_words)]

        def out_slice(c):
            return out_hbm.at[pl.ds((row0 + c * chunk_rows) * n_sampled,
                                    out_words)]

        pltpu.async_copy(in_slice(0), in_bufs[0], isems[0])
        pltpu.async_copy(in_slice(1), in_bufs[1], isems[1])

        def outer(p, _):
            for b in range(2):
                c = 2 * p + b
                in_b, out_b = in_bufs[b], out_bufs[b]
                pltpu.make_async_copy(in_slice(c), in_b, isems[b]).wait()

                @pl.when(p > 0)
                def _():
                    pltpu.make_async_copy(out_b, out_slice(c),
                                          osems[b]).wait()

                @plsc.parallel_loop(0, chunk_rows)
                def _(r):
                    rb = r * ny
                    ob = r * n_sampled
                    for g in range(ngroups):
                        v = plsc.load_gather(in_b, [cols[g] + rb])
                        out_b[pl.ds(ob + g * _L, _L)] = v

                pltpu.async_copy(out_b, out_slice(c), osems[b])

                @pl.when(p < nchunks // 2 - 1)
                def _():
                    pltpu.async_copy(in_slice(c + 2), in_b, isems[b])
            return 0
        lax.fori_loop(0, nchunks // 2, outer, 0)

        for b in range(2):
            pltpu.make_async_copy(out_bufs[b],
                                  out_slice(nchunks - 2 + b),
                                  osems[b]).wait()

    out = run(x_flat, mask_i32)
    return out.reshape(nb, nc, nx, n_sampled)


# speedup vs baseline: 2.9305x; 2.0971x over previous
import functools

import jax
import jax.numpy as jnp
from jax import lax
from jax.experimental import pallas as pl
from jax.experimental.pallas import tpu as pltpu
from jax.experimental.pallas import tpu_sc as plsc

_NCORES = 2
_NSUB = 16
_NW = _NCORES * _NSUB
_L = 16


def kernel(kspace_data, mask):
    nb, nc, nx, ny = kspace_data.shape
    n_sampled = mask.shape[2] // 2
    nrows = nb * nc * nx
    rows_per_w = nrows // _NW
    chunk_rows = 40
    nchunks = rows_per_w // chunk_rows
    ngroups = n_sampled // _L
    assert nchunks % 2 == 0

    x2 = kspace_data.reshape(nrows, ny)
    mask_i32 = mask.reshape(ny).astype(jnp.int32)

    mesh = plsc.VectorSubcoreMesh(
        core_axis_name="c", subcore_axis_name="s",
        num_cores=_NCORES, num_subcores=_NSUB)

    @functools.partial(
        pl.kernel,
        out_type=jax.ShapeDtypeStruct((nb, nc, nx, n_sampled), jnp.float32),
        mesh=mesh,
        scratch_types=[
            pltpu.VMEM((chunk_rows, ny), jnp.float32),
            pltpu.VMEM((chunk_rows, ny), jnp.float32),
            pltpu.VMEM((chunk_rows, n_sampled), jnp.float32),
            pltpu.VMEM((chunk_rows, n_sampled), jnp.float32),
            pltpu.VMEM((ny,), jnp.int32),
            pltpu.VMEM((ny + _L,), jnp.int32),
            pltpu.VMEM((_L,), jnp.int32),
            pltpu.SemaphoreType.DMA,
            pltpu.SemaphoreType.DMA,
            pltpu.SemaphoreType.DMA,
            pltpu.SemaphoreType.DMA,
        ],
        compiler_params=pltpu.CompilerParams(needs_layout_passes=False),
    )
    def run(x_hbm, mask_hbm, out_hbm, in_v0, in_v1, out_v0, out_v1,
            mask_v, cols_v, tmp_v, isem0, isem1, osem0, osem1):
        in_bufs = (in_v0, in_v1)
        out_bufs = (out_v0, out_v1)
        isems = (isem0, isem1)
        osems = (osem0, osem1)

        wid = lax.axis_index("s") * _NCORES + lax.axis_index("c")
        pltpu.sync_copy(mask_hbm, mask_v)

        zero = jnp.zeros((_L,), jnp.int32)
        for g in range(ny // _L + 1):
            cols_v[pl.ds(g * _L, _L)] = zero

        lane = lax.iota(jnp.int32, _L)

        def comp_body(g, off):
            m = mask_v[pl.ds(g * _L, _L)] != 0
            s = jnp.where(m, jnp.int32(1), jnp.int32(0))
            for sh in (1, 2, 4, 8):
                tmp_v[...] = s
                sv = plsc.load_gather(tmp_v, [jnp.maximum(lane - sh, 0)])
                s = s + jnp.where(lane >= sh, sv, jnp.int32(0))
            tmp_v[...] = s
            tot = plsc.load_gather(tmp_v, [jnp.full((_L,), _L - 1,
                                                    jnp.int32)])
            ids = lane + g * _L
            plsc.store_scatter(cols_v, [off + s - 1], ids, mask=m)
            return off + tot
        lax.fori_loop(0, ny // _L, comp_body, jnp.zeros((_L,), jnp.int32))

        cols = [cols_v[pl.ds(g * _L, _L)] for g in range(ngroups)]

        row0 = wid * rows_per_w

        def in_slice(c):
            return x_hbm.at[pl.ds(row0 + c * chunk_rows, chunk_rows), :]

        chunks_per_plane = nx // chunk_rows

        def out_slice(c):
            plane = (rows_per_w // nx) * wid + c // chunks_per_plane
            x0 = (c % chunks_per_plane) * chunk_rows
            return out_hbm.at[plane // nc, plane % nc,
                              pl.ds(x0, chunk_rows), :]

        pltpu.async_copy(in_slice(0), in_bufs[0], isems[0])
        pltpu.async_copy(in_slice(1), in_bufs[1], isems[1])

        def outer(p, _):
            for b in range(2):
                c = 2 * p + b
                in_b, out_b = in_bufs[b], out_bufs[b]
                pltpu.make_async_copy(in_slice(c), in_b, isems[b]).wait()

                @pl.when(p > 0)
                def _():
                    pltpu.make_async_copy(out_b, out_slice(c),
                                          osems[b]).wait()

                @plsc.parallel_loop(0, chunk_rows)
                def _(r):
                    rv = jnp.full((_L,), 0, jnp.int32) + r
                    for g in range(ngroups):
                        v = plsc.load_gather(in_b, [rv, cols[g]])
                        out_b[r, pl.ds(g * _L, _L)] = v

                pltpu.async_copy(out_b, out_slice(c), osems[b])

                @pl.when(p < nchunks // 2 - 1)
                def _():
                    pltpu.async_copy(in_slice(c + 2), in_b, isems[b])
            return 0
        lax.fori_loop(0, nchunks // 2, outer, 0)

        for b in range(2):
            pltpu.make_async_copy(out_bufs[b],
                                  out_slice(nchunks - 2 + b),
                                  osems[b]).wait()

    return run(x2, mask_i32)
